# baseline (speedup 1.0000x reference)
import functools

import jax
import jax.numpy as jnp
from jax import lax
from jax.experimental import pallas as pl
from jax.experimental.pallas import tpu as pltpu
from jax.experimental.pallas import tpu_sc as plsc

_N = 10000
_D = 128
_E = 320000
_NEXP = 8
_NW = 32
_RW = 80
_R = _NW * _RW
_EP = _R * 128
_CH = 64
_CW = _EP // _NW // _CH
_CWF = 300
_CWS = 20
_HF = _CWF // 2
_HS = _CWS // 2
_RPAD = 5248
_NP = 10240
_ZR = _NP // 16
_DS = _NP // 16

_BN = 2048


def _segsum_body(h_hbm, src_hbm, dst_hbm, zeros_hbm, out_hbm, degall_hbm,
                 src_v, dst_v, buf_a, buf_b, deg_v, agg_sh,
                 gsem_a, gsem_b, ssem_a, ssem_b, *, with_deg):
    cid = lax.axis_index("c")
    sid = lax.axis_index("s")
    wid = sid * 2 + cid

    pltpu.sync_copy(zeros_hbm, agg_sh.at[pl.ds(sid * _ZR, _ZR)])
    fast = cid == 0
    halfn = jnp.where(fast, _HF, _HS)
    if with_deg:
        z16 = jnp.zeros((16,), jnp.float32)

        def zbody(i, carry):
            deg_v[pl.ds(i * 16, 16)] = z16
            return carry

        lax.fori_loop(0, _NP // 16, zbody, 0)
    plsc.subcore_barrier()

    bufs = ((buf_a, gsem_a, ssem_a), (buf_b, gsem_b, ssem_b))
    for half in range(2):
        base = jnp.where(fast, sid * _CWF + half * _HF,
                         16 * _CWF + sid * _CWS + half * _HS)
        pltpu.sync_copy(src_hbm.at[pl.ds(base, _HF)], src_v)
        pltpu.sync_copy(dst_hbm.at[pl.ds(base, _HF)], dst_v)
        for b, (buf, gsem, ssem) in enumerate(bufs):
            pltpu.async_copy(h_hbm.at[src_v.at[b]], buf, gsem)
        for b, (buf, gsem, ssem) in enumerate(bufs):
            pltpu.make_async_copy(h_hbm.at[src_v.at[b]], buf, gsem).wait()
            pltpu.async_copy(buf, agg_sh.at[dst_v.at[b]], ssem, add=True)

        def body(i, carry):
            for b, (buf, gsem, ssem) in enumerate(bufs):
                jj = 2 * i + b
                pltpu.make_async_copy(buf, agg_sh.at[dst_v.at[jj - 2]],
                                      ssem).wait()
                pltpu.async_copy(h_hbm.at[src_v.at[jj]], buf, gsem)
                pltpu.make_async_copy(h_hbm.at[src_v.at[jj]], buf,
                                      gsem).wait()
                pltpu.async_copy(buf, agg_sh.at[dst_v.at[jj]], ssem,
                                 add=True)
            return carry

        lax.fori_loop(1, halfn // 2, body, 0)
        for b, (buf, gsem, ssem) in enumerate(bufs):
            pltpu.make_async_copy(buf, agg_sh.at[dst_v.at[0]], ssem).wait()

        if with_deg:
            ones16 = jnp.full((16,), 1.0, jnp.float32)

            def deg_body(j, carry):
                for c in range(_CH // 16):
                    idx = dst_v[j, pl.ds(c * 16, 16)]
                    plsc.addupdate_scatter(deg_v, [idx], ones16)
                return carry

            lax.fori_loop(0, halfn, deg_body, 0)
    if with_deg:
        pltpu.sync_copy(deg_v, degall_hbm.at[pl.ds(wid * _NP, _NP)])
    plsc.subcore_barrier()

    pltpu.sync_copy(agg_sh.at[pl.ds(sid * _ZR, _ZR)],
                    out_hbm.at[cid, pl.ds(sid * _ZR, _ZR)])


def _make_segsum(with_deg):
    mesh = plsc.VectorSubcoreMesh(core_axis_name="c", subcore_axis_name="s")
    if with_deg:
        out_type = (jax.ShapeDtypeStruct((2, _NP, _D), jnp.float32),
                    jax.ShapeDtypeStruct((_NW * _NP,), jnp.float32))
        body = functools.partial(_segsum_body, with_deg=True)
    else:
        out_type = jax.ShapeDtypeStruct((2, _NP, _D), jnp.float32)

        def body(h, s, dd, z, out, src_v, dst_v, buf_a, buf_b, deg_v,
                 agg_sh, gsem_a, gsem_b, ssem_a, ssem_b):
            _segsum_body(h, s, dd, z, out, None, src_v, dst_v, buf_a, buf_b,
                         deg_v, agg_sh, gsem_a, gsem_b, ssem_a, ssem_b,
                         with_deg=False)
    return pl.kernel(
        body,
        out_type=out_type,
        mesh=mesh,
        compiler_params=pltpu.CompilerParams(needs_layout_passes=False,
                                             use_tc_tiling_on_sc=False),
        scratch_types=[
            pltpu.VMEM((_HF, _CH), jnp.int32),
            pltpu.VMEM((_HF, _CH), jnp.int32),
            pltpu.VMEM((_CH, _D), jnp.float32),
            pltpu.VMEM((_CH, _D), jnp.float32),
            pltpu.VMEM((_NP,), jnp.float32),
            pltpu.VMEM_SHARED((_NP, _D), jnp.float32),
            pltpu.SemaphoreType.DMA,
            pltpu.SemaphoreType.DMA,
            pltpu.SemaphoreType.DMA,
            pltpu.SemaphoreType.DMA,
        ],
    )


@functools.lru_cache(maxsize=None)
def _get_segsum(with_deg):
    return _make_segsum(with_deg)


def _moe_math(parts_ref, d_ref, wg_ref, we_ref, be_ref):
    p = parts_ref[0] + parts_ref[1]
    deg16 = jnp.sum(d_ref[...], axis=0)
    inv16 = 1.0 / jnp.maximum(deg16, 1.0)
    row = lax.broadcasted_iota(jnp.int32, (p.shape[0], _D), 0)
    col = lax.broadcasted_iota(jnp.int32, (p.shape[0], _D), 1)
    expanded = jnp.zeros_like(p)
    for r in range(16):
        expanded = expanded + jnp.where(row // _D == r, inv16[r:r + 1, :], 0.0)
    inv_col = jnp.sum(jnp.where(col == row % _D, expanded, 0.0), axis=1,
                      keepdims=True)
    agg = p * inv_col
    logits = lax.dot_general(agg, wg_ref[...], (((1,), (0,)), ((), ())),
                             preferred_element_type=jnp.float32)
    neg = jnp.float32(-1e30)
    logits = jnp.where(col < _NEXP, logits, neg)
    cur = logits
    ms, hs = [], []
    for _ in range(3):
        m = jnp.max(cur, axis=1, keepdims=True)
        am = jnp.min(jnp.where(cur == m, col, 10**6), axis=1, keepdims=True)
        hs.append((col == am).astype(jnp.float32))
        ms.append(m)
        cur = jnp.where(col == am, neg, cur)
    e2 = jnp.exp(ms[1] - ms[0])
    e3 = jnp.exp(ms[2] - ms[0])
    z = 1.0 + e2 + e3
    w = (hs[0] + hs[1] * e2 + hs[2] * e3) / z
    def b16(v):
        return v.astype(jnp.bfloat16).astype(jnp.float32)

    acc = jnp.zeros_like(p)
    for e in range(_NEXP):
        eo = lax.dot_general(agg, we_ref[e], (((1,), (0,)), ((), ())),
                             preferred_element_type=jnp.float32)
        eo = eo + be_ref[e:e + 1, :]
        acc = acc + b16(eo) * b16(w[:, e:e + 1])
    return jnp.maximum(acc, 0.0)


def _moe_mid_kernel(parts_ref, d_ref, wg_ref, we_ref, be_ref, out_ref):
    out_ref[...] = _moe_math(parts_ref, d_ref, wg_ref, we_ref, be_ref)


def _moe_last_kernel(parts_ref, d_ref, wg_ref, we_ref, be_ref,
                     wout_ref, bout_ref, out_ref):
    h = _moe_math(parts_ref, d_ref, wg_ref, we_ref, be_ref)
    o = lax.dot_general(h, wout_ref[...], (((1,), (0,)), ((), ())),
                        preferred_element_type=jnp.float32)
    out_ref[...] = o + bout_ref[...]


_COMMON_SPECS = [
    pl.BlockSpec((2, _BN, _D), lambda i: (0, i, 0)),
    pl.BlockSpec((_NW, _BN // _D, _D), lambda i: (0, i, 0)),
    pl.BlockSpec((_D, _D), lambda i: (0, 0)),
    pl.BlockSpec((_NEXP, _D, _D), lambda i: (0, 0, 0)),
    pl.BlockSpec((_D, _D), lambda i: (0, 0)),
]


def _moe_mid(parts, d, wgp, we, bep):
    return pl.pallas_call(
        _moe_mid_kernel,
        grid=(_NP // _BN,),
        in_specs=_COMMON_SPECS,
        out_specs=pl.BlockSpec((_BN, _D), lambda i: (i, 0)),
        out_shape=jax.ShapeDtypeStruct((_NP, _D), jnp.float32),
    )(parts, d, wgp, we, bep)


def _moe_last(parts, d, wgp, we, bep, wout, bout):
    return pl.pallas_call(
        _moe_last_kernel,
        grid=(_NP // _BN,),
        in_specs=_COMMON_SPECS + [
            pl.BlockSpec((_D, _D), lambda i: (0, 0)),
            pl.BlockSpec((1, _D), lambda i: (0, 0)),
        ],
        out_specs=pl.BlockSpec((_BN, _D), lambda i: (i, 0)),
        out_shape=jax.ShapeDtypeStruct((_N, _D), jnp.float32),
    )(parts, d, wgp, we, bep, wout, bout)


def kernel(x, edge_index, Wg0, We0, be0, Wg1, We1, be1, Wout, bout):
    src = edge_index[0].astype(jnp.int32)
    dst = edge_index[1].astype(jnp.int32)
    pad = _RPAD * _CH - _E
    src_m = jnp.concatenate([src, jnp.zeros((pad,), jnp.int32)]).reshape(-1, _CH)
    dst_m = jnp.concatenate([dst, jnp.full((pad,), _N, jnp.int32)]).reshape(-1, _CH)
    zeros_blk = jnp.zeros((_ZR, _D), jnp.float32)
    wg0p = jnp.pad(Wg0, ((0, 0), (0, _D - _NEXP)))
    wg1p = jnp.pad(Wg1, ((0, 0), (0, _D - _NEXP)))
    be0p = jnp.pad(be0, ((0, _D - _NEXP), (0, 0)))
    be1p = jnp.pad(be1, ((0, _D - _NEXP), (0, 0)))

    parts0, degall = _get_segsum(True)(x, src_m, dst_m, zeros_blk)
    d = degall.reshape(_NW, _NP // _D, _D)
    h1 = _moe_mid(parts0, d, wg0p, We0, be0p)
    parts1 = _get_segsum(False)(h1, src_m, dst_m, zeros_blk)
    return _moe_last(parts1, d, wg1p, We1, be1p, Wout, bout[None, :])


# speedup vs baseline: 1.0775x; 1.0775x over previous
import functools

import jax
import jax.numpy as jnp
from jax import lax
from jax.experimental import pallas as pl
from jax.experimental.pallas import tpu as pltpu
from jax.experimental.pallas import tpu_sc as plsc

_N = 10000
_D = 128
_E = 320000
_NEXP = 8
_NW = 32
_RW = 80
_R = _NW * _RW
_EP = _R * 128
_CH = 64
_CW = _EP // _NW // _CH
_CWF = 288
_CWS = 32
_HF = _CWF // 2
_HS = _CWS // 2
_RPAD = 5248
_NP = 10240
_ZR = _NP // 16
_DS = _NP // 16

_BN = 2048


def _segsum_body(h_hbm, src_hbm, dst_hbm, zeros_hbm, out_hbm, degall_hbm,
                 src_v, dst_v, buf_a, buf_b, deg_v, agg_sh,
                 gsem_a, gsem_b, ssem_a, ssem_b, *, with_deg):
    cid = lax.axis_index("c")
    sid = lax.axis_index("s")
    wid = sid * 2 + cid

    pltpu.sync_copy(zeros_hbm, agg_sh.at[pl.ds(sid * _ZR, _ZR)])
    fast = cid == 0
    halfn = jnp.where(fast, _HF, _HS)
    if with_deg:
        z16 = jnp.zeros((16,), jnp.float32)

        def zbody(i, carry):
            deg_v[pl.ds(i * 16, 16)] = z16
            return carry

        lax.fori_loop(0, _NP // 16, zbody, 0)
    plsc.subcore_barrier()

    bufs = ((buf_a, gsem_a, ssem_a), (buf_b, gsem_b, ssem_b))
    for half in range(2):
        base = jnp.where(fast, sid * _CWF + half * _HF,
                         16 * _CWF + sid * _CWS + half * _HS)
        pltpu.sync_copy(src_hbm.at[pl.ds(base, _HF)], src_v)
        pltpu.sync_copy(dst_hbm.at[pl.ds(base, _HF)], dst_v)
        for b, (buf, gsem, ssem) in enumerate(bufs):
            pltpu.async_copy(h_hbm.at[src_v.at[b]], buf, gsem)
        for b, (buf, gsem, ssem) in enumerate(bufs):
            pltpu.make_async_copy(h_hbm.at[src_v.at[b]], buf, gsem).wait()
            pltpu.async_copy(buf, agg_sh.at[dst_v.at[b]], ssem, add=True)

        def body(i, carry):
            for b, (buf, gsem, ssem) in enumerate(bufs):
                jj = 2 * i + b
                pltpu.make_async_copy(buf, agg_sh.at[dst_v.at[jj - 2]],
                                      ssem).wait()
                pltpu.async_copy(h_hbm.at[src_v.at[jj]], buf, gsem)
                pltpu.make_async_copy(h_hbm.at[src_v.at[jj]], buf,
                                      gsem).wait()
                pltpu.async_copy(buf, agg_sh.at[dst_v.at[jj]], ssem,
                                 add=True)
            return carry

        lax.fori_loop(1, halfn // 2, body, 0)
        for b, (buf, gsem, ssem) in enumerate(bufs):
            pltpu.make_async_copy(buf, agg_sh.at[dst_v.at[0]], ssem).wait()

        if with_deg:
            ones16 = jnp.full((16,), 1.0, jnp.float32)

            def deg_body(j, carry):
                for c in range(_CH // 16):
                    idx = dst_v[j, pl.ds(c * 16, 16)]
                    plsc.addupdate_scatter(deg_v, [idx], ones16)
                return carry

            lax.fori_loop(0, halfn, deg_body, 0)
    if with_deg:
        pltpu.sync_copy(deg_v, degall_hbm.at[pl.ds(wid * _NP, _NP)])
    plsc.subcore_barrier()

    pltpu.sync_copy(agg_sh.at[pl.ds(sid * _ZR, _ZR)],
                    out_hbm.at[cid, pl.ds(sid * _ZR, _ZR)])


def _make_segsum(with_deg):
    mesh = plsc.VectorSubcoreMesh(core_axis_name="c", subcore_axis_name="s")
    if with_deg:
        out_type = (jax.ShapeDtypeStruct((2, _NP, _D), jnp.float32),
                    jax.ShapeDtypeStruct((_NW * _NP,), jnp.float32))
        body = functools.partial(_segsum_body, with_deg=True)
    else:
        out_type = jax.ShapeDtypeStruct((2, _NP, _D), jnp.float32)

        def body(h, s, dd, z, out, src_v, dst_v, buf_a, buf_b, deg_v,
                 agg_sh, gsem_a, gsem_b, ssem_a, ssem_b):
            _segsum_body(h, s, dd, z, out, None, src_v, dst_v, buf_a, buf_b,
                         deg_v, agg_sh, gsem_a, gsem_b, ssem_a, ssem_b,
                         with_deg=False)
    return pl.kernel(
        body,
        out_type=out_type,
        mesh=mesh,
        compiler_params=pltpu.CompilerParams(needs_layout_passes=False,
                                             use_tc_tiling_on_sc=False),
        scratch_types=[
            pltpu.VMEM((_HF, _CH), jnp.int32),
            pltpu.VMEM((_HF, _CH), jnp.int32),
            pltpu.VMEM((_CH, _D), jnp.float32),
            pltpu.VMEM((_CH, _D), jnp.float32),
            pltpu.VMEM((_NP,), jnp.float32),
            pltpu.VMEM_SHARED((_NP, _D), jnp.float32),
            pltpu.SemaphoreType.DMA,
            pltpu.SemaphoreType.DMA,
            pltpu.SemaphoreType.DMA,
            pltpu.SemaphoreType.DMA,
        ],
    )


@functools.lru_cache(maxsize=None)
def _get_segsum(with_deg):
    return _make_segsum(with_deg)


def _moe_math(parts_ref, d_ref, wg_ref, we_ref, be_ref):
    p = parts_ref[0] + parts_ref[1]
    deg16 = jnp.sum(d_ref[...], axis=0)
    inv16 = 1.0 / jnp.maximum(deg16, 1.0)
    row = lax.broadcasted_iota(jnp.int32, (p.shape[0], _D), 0)
    col = lax.broadcasted_iota(jnp.int32, (p.shape[0], _D), 1)
    expanded = jnp.zeros_like(p)
    for r in range(16):
        expanded = expanded + jnp.where(row // _D == r, inv16[r:r + 1, :], 0.0)
    inv_col = jnp.sum(jnp.where(col == row % _D, expanded, 0.0), axis=1,
                      keepdims=True)
    agg = p * inv_col
    logits = lax.dot_general(agg, wg_ref[...], (((1,), (0,)), ((), ())),
                             preferred_element_type=jnp.float32)
    neg = jnp.float32(-1e30)
    logits = jnp.where(col < _NEXP, logits, neg)
    cur = logits
    ms, hs = [], []
    for _ in range(3):
        m = jnp.max(cur, axis=1, keepdims=True)
        am = jnp.min(jnp.where(cur == m, col, 10**6), axis=1, keepdims=True)
        hs.append((col == am).astype(jnp.float32))
        ms.append(m)
        cur = jnp.where(col == am, neg, cur)
    e2 = jnp.exp(ms[1] - ms[0])
    e3 = jnp.exp(ms[2] - ms[0])
    z = 1.0 + e2 + e3
    w = (hs[0] + hs[1] * e2 + hs[2] * e3) / z
    def b16(v):
        return v.astype(jnp.bfloat16).astype(jnp.float32)

    acc = jnp.zeros_like(p)
    for e in range(_NEXP):
        eo = lax.dot_general(agg, we_ref[e], (((1,), (0,)), ((), ())),
                             preferred_element_type=jnp.float32)
        eo = eo + be_ref[e:e + 1, :]
        acc = acc + b16(eo) * b16(w[:, e:e + 1])
    return jnp.maximum(acc, 0.0)


def _moe_mid_kernel(parts_ref, d_ref, wg_ref, we_ref, be_ref, out_ref):
    out_ref[...] = _moe_math(parts_ref, d_ref, wg_ref, we_ref, be_ref)


def _moe_last_kernel(parts_ref, d_ref, wg_ref, we_ref, be_ref,
                     wout_ref, bout_ref, out_ref):
    h = _moe_math(parts_ref, d_ref, wg_ref, we_ref, be_ref)
    o = lax.dot_general(h, wout_ref[...], (((1,), (0,)), ((), ())),
                        preferred_element_type=jnp.float32)
    out_ref[...] = o + bout_ref[...]


_COMMON_SPECS = [
    pl.BlockSpec((2, _BN, _D), lambda i: (0, i, 0)),
    pl.BlockSpec((_NW, _BN // _D, _D), lambda i: (0, i, 0)),
    pl.BlockSpec((_D, _D), lambda i: (0, 0)),
    pl.BlockSpec((_NEXP, _D, _D), lambda i: (0, 0, 0)),
    pl.BlockSpec((_D, _D), lambda i: (0, 0)),
]


def _moe_mid(parts, d, wgp, we, bep):
    return pl.pallas_call(
        _moe_mid_kernel,
        grid=(_NP // _BN,),
        in_specs=_COMMON_SPECS,
        out_specs=pl.BlockSpec((_BN, _D), lambda i: (i, 0)),
        out_shape=jax.ShapeDtypeStruct((_NP, _D), jnp.float32),
    )(parts, d, wgp, we, bep)


def _moe_last(parts, d, wgp, we, bep, wout, bout):
    return pl.pallas_call(
        _moe_last_kernel,
        grid=(_NP // _BN,),
        in_specs=_COMMON_SPECS + [
            pl.BlockSpec((_D, _D), lambda i: (0, 0)),
            pl.BlockSpec((1, _D), lambda i: (0, 0)),
        ],
        out_specs=pl.BlockSpec((_BN, _D), lambda i: (i, 0)),
        out_shape=jax.ShapeDtypeStruct((_N, _D), jnp.float32),
    )(parts, d, wgp, we, bep, wout, bout)


def kernel(x, edge_index, Wg0, We0, be0, Wg1, We1, be1, Wout, bout):
    src = edge_index[0].astype(jnp.int32)
    dst = edge_index[1].astype(jnp.int32)
    pad = _RPAD * _CH - _E
    src_m = jnp.concatenate([src, jnp.zeros((pad,), jnp.int32)]).reshape(-1, _CH)
    dst_m = jnp.concatenate([dst, jnp.full((pad,), _N, jnp.int32)]).reshape(-1, _CH)
    zeros_blk = jnp.zeros((_ZR, _D), jnp.float32)
    wg0p = jnp.pad(Wg0, ((0, 0), (0, _D - _NEXP)))
    wg1p = jnp.pad(Wg1, ((0, 0), (0, _D - _NEXP)))
    be0p = jnp.pad(be0, ((0, _D - _NEXP), (0, 0)))
    be1p = jnp.pad(be1, ((0, _D - _NEXP), (0, 0)))

    parts0, degall = _get_segsum(True)(x, src_m, dst_m, zeros_blk)
    d = degall.reshape(_NW, _NP // _D, _D)
    h1 = _moe_mid(parts0, d, wg0p, We0, be0p)
    parts1 = _get_segsum(False)(h1, src_m, dst_m, zeros_blk)
    return _moe_last(parts1, d, wg1p, We1, be1p, Wout, bout[None, :])
